# baseline (speedup 1.0000x reference)
import functools

import jax
import jax.numpy as jnp
from jax import lax
from jax.experimental import pallas as pl
from jax.experimental.pallas import tpu as pltpu
from jax.experimental.pallas import tpu_sc as plsc

EMBED_DIM = 64
NUM_CORES = 2
NUM_SUBCORES = 16
NUM_WORKERS = NUM_CORES * NUM_SUBCORES
RPG = 4
NBUF = 4


def _gather_call(batch: int, hist: int):
    mesh = plsc.VectorSubcoreMesh(core_axis_name="c", subcore_axis_name="s")
    rows_per_w = batch // NUM_WORKERS
    n_grp = rows_per_w // RPG

    @functools.partial(
        pl.kernel,
        out_type=jax.ShapeDtypeStruct((batch, hist, EMBED_DIM), jnp.float32),
        mesh=mesh,
        scratch_types=[
            pltpu.VMEM((rows_per_w, hist), jnp.int32),
            pltpu.VMEM((NBUF, RPG, hist, EMBED_DIM), jnp.float32),
            pltpu.SemaphoreType.DMA((NBUF,)),
            pltpu.SemaphoreType.DMA((NBUF,)),
        ],
        compiler_params=pltpu.CompilerParams(use_tc_tiling_on_sc=False),
    )
    def k(idx_hbm, table_hbm, out_hbm, idx_v, rows_v, gsem, osem):
        wid = lax.axis_index("s") * NUM_CORES + lax.axis_index("c")
        base = wid * rows_per_w

        pltpu.sync_copy(idx_hbm.at[pl.ds(base, rows_per_w), :], idx_v)

        def gather_grp(g, b):
            for i in range(RPG):
                pltpu.async_copy(
                    table_hbm.at[idx_v.at[g * RPG + i, :]],
                    rows_v.at[b, i],
                    gsem.at[b],
                )

        def gather_wait(g, b):
            for i in range(RPG):
                pltpu.make_async_copy(
                    table_hbm.at[idx_v.at[g * RPG + i, :]],
                    rows_v.at[b, i],
                    gsem.at[b],
                ).wait()

        def out_start(g, b):
            pltpu.async_copy(
                rows_v.at[b], out_hbm.at[pl.ds(base + g * RPG, RPG), :, :],
                osem.at[b],
            )

        def out_wait(g, b):
            pltpu.make_async_copy(
                rows_v.at[b], out_hbm.at[pl.ds(base + g * RPG, RPG), :, :],
                osem.at[b],
            ).wait()

        for b in range(NBUF):
            gather_grp(b, b)

        def outer(o, carry):
            for b in range(NBUF):
                g = o * NBUF + b
                gather_wait(g, b)
                out_start(g, b)

                @pl.when(o < n_grp // NBUF - 1)
                def _():
                    out_wait(g, b)
                    gather_grp(g + NBUF, b)

            return carry

        lax.fori_loop(0, n_grp // NBUF, outer, 0)

        for b in range(NBUF):
            out_wait(n_grp - NBUF + b, b)

    return k


def kernel(x, embedding):
    batch, hist = x.shape
    out = _gather_call(batch, hist)(x.astype(jnp.int32), embedding)
    return out


# speedup vs baseline: 1.0021x; 1.0021x over previous
import functools

import jax
import jax.numpy as jnp
from jax import lax
from jax.experimental import pallas as pl
from jax.experimental.pallas import tpu as pltpu
from jax.experimental.pallas import tpu_sc as plsc

EMBED_DIM = 64
NUM_CORES = 2
NUM_SUBCORES = 16
NUM_WORKERS = NUM_CORES * NUM_SUBCORES
CHUNK = 128
NBUF = 4


def _gather_call(batch: int, hist: int):
    mesh = plsc.VectorSubcoreMesh(core_axis_name="c", subcore_axis_name="s")
    b_per_w = batch // NUM_WORKERS
    n_col = b_per_w // CHUNK
    n_step = hist * n_col

    @functools.partial(
        pl.kernel,
        out_type=jax.ShapeDtypeStruct((batch, hist, EMBED_DIM), jnp.float32),
        mesh=mesh,
        scratch_types=[
            pltpu.VMEM((hist, b_per_w), jnp.int32),
            pltpu.VMEM((NBUF, CHUNK, EMBED_DIM), jnp.float32),
            pltpu.SemaphoreType.DMA((NBUF,)),
            pltpu.SemaphoreType.DMA((NBUF,)),
        ],
        compiler_params=pltpu.CompilerParams(use_tc_tiling_on_sc=False),
    )
    def k(idx_hbm, table_hbm, out_hbm, idx_v, rows_v, gsem, osem):
        wid = lax.axis_index("s") * NUM_CORES + lax.axis_index("c")
        base = wid * b_per_w

        pltpu.sync_copy(idx_hbm.at[:, pl.ds(base, b_per_w)], idx_v)

        def gather_start(j, b):
            t = j // n_col
            c0 = (j % n_col) * CHUNK
            pltpu.async_copy(
                table_hbm.at[idx_v.at[t, pl.ds(c0, CHUNK)]],
                rows_v.at[b],
                gsem.at[b],
            )

        def gather_wait(j, b):
            t = j // n_col
            c0 = (j % n_col) * CHUNK
            pltpu.make_async_copy(
                table_hbm.at[idx_v.at[t, pl.ds(c0, CHUNK)]],
                rows_v.at[b],
                gsem.at[b],
            ).wait()

        def out_start(j, b):
            t = j // n_col
            c0 = (j % n_col) * CHUNK
            pltpu.async_copy(
                rows_v.at[b],
                out_hbm.at[pl.ds(base + c0, CHUNK), t, :],
                osem.at[b],
            )

        def out_wait(j, b):
            t = j // n_col
            c0 = (j % n_col) * CHUNK
            pltpu.make_async_copy(
                rows_v.at[b],
                out_hbm.at[pl.ds(base + c0, CHUNK), t, :],
                osem.at[b],
            ).wait()

        for b in range(NBUF):
            gather_start(b, b)

        def outer(o, carry):
            for b in range(NBUF):
                j = o * NBUF + b
                gather_wait(j, b)
                out_start(j, b)

                @pl.when(o < n_step // NBUF - 1)
                def _():
                    out_wait(j, b)
                    gather_start(j + NBUF, b)

            return carry

        lax.fori_loop(0, n_step // NBUF, outer, 0)

        for b in range(NBUF):
            out_wait(n_step - NBUF + b, b)

    return k


def kernel(x, embedding):
    batch, hist = x.shape
    xt = x.T.astype(jnp.int32)
    return _gather_call(batch, hist)(xt, embedding)


# speedup vs baseline: 1.0062x; 1.0041x over previous
import functools

import jax
import jax.numpy as jnp
from jax import lax
from jax.experimental import pallas as pl
from jax.experimental.pallas import tpu as pltpu
from jax.experimental.pallas import tpu_sc as plsc

EMBED_DIM = 64
NUM_CORES = 2
NUM_SUBCORES = 16
NUM_WORKERS = NUM_CORES * NUM_SUBCORES
CHUNK = 256
NBUF = 4


def _gather_call(batch: int, hist: int):
    mesh = plsc.VectorSubcoreMesh(core_axis_name="c", subcore_axis_name="s")
    b_per_w = batch // NUM_WORKERS
    n_col = b_per_w // CHUNK
    n_step = hist * n_col

    @functools.partial(
        pl.kernel,
        out_type=jax.ShapeDtypeStruct((batch, hist, EMBED_DIM), jnp.float32),
        mesh=mesh,
        scratch_types=[
            pltpu.VMEM((hist, b_per_w), jnp.int32),
            pltpu.VMEM((NBUF, CHUNK, EMBED_DIM), jnp.float32),
            pltpu.SemaphoreType.DMA((NBUF,)),
            pltpu.SemaphoreType.DMA((NBUF,)),
        ],
        compiler_params=pltpu.CompilerParams(use_tc_tiling_on_sc=False),
    )
    def k(idx_hbm, table_hbm, out_hbm, idx_v, rows_v, gsem, osem):
        wid = lax.axis_index("s") * NUM_CORES + lax.axis_index("c")
        base = wid * b_per_w

        pltpu.sync_copy(idx_hbm.at[:, pl.ds(base, b_per_w)], idx_v)

        def gather_start(j, b):
            t = j // n_col
            c0 = (j % n_col) * CHUNK
            pltpu.async_copy(
                table_hbm.at[idx_v.at[t, pl.ds(c0, CHUNK)]],
                rows_v.at[b],
                gsem.at[b],
            )

        def gather_wait(j, b):
            t = j // n_col
            c0 = (j % n_col) * CHUNK
            pltpu.make_async_copy(
                table_hbm.at[idx_v.at[t, pl.ds(c0, CHUNK)]],
                rows_v.at[b],
                gsem.at[b],
            ).wait()

        def out_start(j, b):
            t = j // n_col
            c0 = (j % n_col) * CHUNK
            pltpu.async_copy(
                rows_v.at[b],
                out_hbm.at[pl.ds(base + c0, CHUNK), t, :],
                osem.at[b],
            )

        def out_wait(j, b):
            t = j // n_col
            c0 = (j % n_col) * CHUNK
            pltpu.make_async_copy(
                rows_v.at[b],
                out_hbm.at[pl.ds(base + c0, CHUNK), t, :],
                osem.at[b],
            ).wait()

        for b in range(NBUF):
            gather_start(b, b)

        def outer(o, carry):
            for b in range(NBUF):
                j = o * NBUF + b
                gather_wait(j, b)
                out_start(j, b)

                @pl.when(o < n_step // NBUF - 1)
                def _():
                    out_wait(j, b)
                    gather_start(j + NBUF, b)

            return carry

        lax.fori_loop(0, n_step // NBUF, outer, 0)

        for b in range(NBUF):
            out_wait(n_step - NBUF + b, b)

    return k


def kernel(x, embedding):
    batch, hist = x.shape
    xt = x.T.astype(jnp.int32)
    return _gather_call(batch, hist)(xt, embedding)
